# baseline (speedup 1.0000x reference)
import functools

import jax
import jax.numpy as jnp
from jax import lax
from jax.experimental import pallas as pl
from jax.experimental.pallas import tpu as pltpu
from jax.experimental.pallas import tpu_sc as plsc

N = 10000
E = 320000
D = 128
NPAD = 10240
NC = 2
NS = 16
NW = NC * NS
C = 128
NCHUNK = E // C
CHUNKS_PER_W = NCHUNK // NW
REMAINDER = NCHUNK - CHUNKS_PER_W * NW
ROWS_PER_S = NPAD // NS

_sc_mesh = plsc.VectorSubcoreMesh(core_axis_name="c", subcore_axis_name="s")


@functools.partial(
    pl.kernel,
    out_type=(
        jax.ShapeDtypeStruct((NC, NPAD, D), jnp.float32),
        jax.ShapeDtypeStruct((NC, NPAD, 16), jnp.float32),
    ),
    mesh=_sc_mesh,
    scratch_types=[
        pltpu.VMEM((2, C), jnp.int32),
        pltpu.VMEM((2, C), jnp.int32),
        pltpu.VMEM((C, D), jnp.float32),
        pltpu.VMEM((C, D), jnp.float32),
        pltpu.VMEM((C, 16), jnp.float32),
        pltpu.VMEM_SHARED((NPAD, D), jnp.float32),
        pltpu.VMEM_SHARED((NPAD, 16), jnp.float32),
        pltpu.SemaphoreType.DMA,
        pltpu.SemaphoreType.DMA,
        pltpu.SemaphoreType.DMA,
        pltpu.SemaphoreType.DMA,
        pltpu.SemaphoreType.DMA,
        pltpu.SemaphoreType.DMA,
        pltpu.SemaphoreType.DMA,
        pltpu.SemaphoreType.DMA,
    ],
    compiler_params=pltpu.CompilerParams(use_tc_tiling_on_sc=False),
)
def _sc_agg(x_hbm, idx_hbm, zrows_hbm, zdeg_hbm, aggf_hbm, dego_hbm,
            idx0, idx1, rows0, rows1, ones, aggsh, degsh,
            semi0, semi1, semg0, semg1, sems0, sems1, semd0, semd1):
    cid = lax.axis_index("c")
    sid = lax.axis_index("s")
    wid = sid * NC + cid
    base = wid * CHUNKS_PER_W

    for j in range(C):
        ones[j, :] = jnp.ones((16,), jnp.float32)

    def start_idx(ci, bank, sem):
        pltpu.async_copy(idx_hbm.at[ci], bank, sem)

    def wait_idx(bank, sem):
        pltpu.make_async_copy(idx_hbm.at[0], bank, sem).wait()

    def start_gather(bank, buf, sem):
        pltpu.async_copy(x_hbm.at[bank.at[0]], buf, sem)

    def wait_gather(buf, sem):
        pltpu.make_async_copy(x_hbm.at[idx0.at[0]], buf, sem).wait()

    def start_scatter(buf, bank, sems, semd):
        pltpu.async_copy(buf, aggsh.at[bank.at[1]], sems, add=True)
        pltpu.async_copy(ones, degsh.at[bank.at[1]], semd, add=True)

    def wait_scatter(buf, sems, semd):
        pltpu.make_async_copy(buf, aggsh.at[idx0.at[1]], sems).wait()
        pltpu.make_async_copy(ones, degsh.at[idx0.at[1]], semd).wait()

    start_idx(base, idx0, semi0)
    start_idx(base + 1, idx1, semi1)
    wait_idx(idx0, semi0)
    start_gather(idx0, rows0, semg0)

    pltpu.sync_copy(zrows_hbm, aggsh.at[pl.ds(sid * ROWS_PER_S, ROWS_PER_S)])
    pltpu.sync_copy(zdeg_hbm, degsh.at[pl.ds(sid * ROWS_PER_S, ROWS_PER_S)])
    plsc.subcore_barrier()

    def _loop_body(k, _):
        c0 = base + 2 * k
        c1 = c0 + 1
        last = k >= CHUNKS_PER_W // 2 - 1
        wait_gather(rows0, semg0)
        start_scatter(rows0, idx0, sems0, semd0)
        wait_idx(idx1, semi1)
        start_gather(idx1, rows1, semg1)
        wait_scatter(rows0, sems0, semd0)

        @pl.when(~last)
        def _():
            start_idx(c0 + 2, idx0, semi0)

        wait_gather(rows1, semg1)
        start_scatter(rows1, idx1, sems1, semd1)

        @pl.when(~last)
        def _():
            wait_idx(idx0, semi0)
            start_gather(idx0, rows0, semg0)

        wait_scatter(rows1, sems1, semd1)

        @pl.when(~last)
        def _():
            start_idx(c1 + 2, idx1, semi1)

        return 0

    lax.fori_loop(0, CHUNKS_PER_W // 2, _loop_body, 0)

    @pl.when(wid < REMAINDER)
    def _():
        ci = NW * CHUNKS_PER_W + wid
        pltpu.sync_copy(idx_hbm.at[ci], idx0)
        pltpu.async_copy(x_hbm.at[idx0.at[0]], rows0, semg0).wait()
        pltpu.sync_copy(rows0, aggsh.at[idx0.at[1]], add=True)
        pltpu.sync_copy(ones, degsh.at[idx0.at[1]], add=True)

    plsc.subcore_barrier()

    pltpu.sync_copy(aggsh.at[pl.ds(sid * ROWS_PER_S, ROWS_PER_S)],
                    aggf_hbm.at[cid, pl.ds(sid * ROWS_PER_S, ROWS_PER_S)])
    pltpu.sync_copy(degsh.at[pl.ds(sid * ROWS_PER_S, ROWS_PER_S)],
                    dego_hbm.at[cid, pl.ds(sid * ROWS_PER_S, ROWS_PER_S)])


_TC_R = 512


def _tc_self_body(x_ref, ws_ref, b_ref, o_ref):
    o_ref[...] = (
        jnp.dot(x_ref[...], ws_ref[...], preferred_element_type=jnp.float32)
        + b_ref[...])


def _tc_self(x, ws, b2):
    return pl.pallas_call(
        _tc_self_body,
        grid=(N // _TC_R + (N % _TC_R > 0),),
        in_specs=[
            pl.BlockSpec((_TC_R, D), lambda i: (i, 0)),
            pl.BlockSpec((D, D), lambda i: (0, 0)),
            pl.BlockSpec((1, D), lambda i: (0, 0)),
        ],
        out_specs=pl.BlockSpec((_TC_R, D), lambda i: (i, 0)),
        out_shape=jax.ShapeDtypeStruct((N, D), jnp.float32),
    )(x, ws, b2)


def _tc_body(ag_ref, deg_ref, h_ref, w_ref, o_ref):
    feat = ag_ref[0] + ag_ref[1]
    deg = deg_ref[0, :, 0:1] + deg_ref[1, :, 0:1]
    m = feat / jnp.maximum(deg, 1.0)
    o_ref[...] = jnp.maximum(
        jnp.dot(m, w_ref[...], preferred_element_type=jnp.float32)
        + h_ref[...],
        0.0,
    )


def _tc_finish(agg, deg, h, w):
    return pl.pallas_call(
        _tc_body,
        grid=(NPAD // _TC_R,),
        in_specs=[
            pl.BlockSpec((NC, _TC_R, D), lambda i: (0, i, 0)),
            pl.BlockSpec((NC, _TC_R, 16), lambda i: (0, i, 0)),
            pl.BlockSpec((_TC_R, D), lambda i: (i, 0)),
            pl.BlockSpec((D, D), lambda i: (0, 0)),
        ],
        out_specs=pl.BlockSpec((_TC_R, D), lambda i: (i, 0)),
        out_shape=jax.ShapeDtypeStruct((N, D), jnp.float32),
    )(agg, deg, h, w)


def kernel(x, edge_index, W, W_self, b):
    idx = edge_index.reshape(2, NCHUNK, C).transpose(1, 0, 2)
    zrows = jnp.zeros((ROWS_PER_S, D), jnp.float32)
    zdeg = jnp.zeros((ROWS_PER_S, 16), jnp.float32)
    h = _tc_self(x, W_self, b.reshape(1, D))
    aggf, dego = _sc_agg(x, idx, zrows, zdeg)
    return _tc_finish(aggf, dego, h, W)


# speedup vs baseline: 1.0019x; 1.0019x over previous
import functools

import jax
import jax.numpy as jnp
from jax import lax
from jax.experimental import pallas as pl
from jax.experimental.pallas import tpu as pltpu
from jax.experimental.pallas import tpu_sc as plsc

N = 10000
E = 320000
D = 128
NPAD = 10240
NC = 2
NS = 16
NW = NC * NS
C = 128
NCHUNK = E // C
CHUNKS_PER_W = NCHUNK // NW
REMAINDER = NCHUNK - CHUNKS_PER_W * NW
ROWS_PER_S = NPAD // NS

_sc_mesh = plsc.VectorSubcoreMesh(core_axis_name="c", subcore_axis_name="s")


@functools.partial(
    pl.kernel,
    out_type=(
        jax.ShapeDtypeStruct((NC, NPAD, D), jnp.float32),
        jax.ShapeDtypeStruct((NC, NPAD, 16), jnp.float32),
    ),
    mesh=_sc_mesh,
    scratch_types=[
        pltpu.VMEM((2, C), jnp.int32),
        pltpu.VMEM((2, C), jnp.int32),
        pltpu.VMEM((C, D), jnp.float32),
        pltpu.VMEM((C, D), jnp.float32),
        pltpu.VMEM((C, 16), jnp.float32),
        pltpu.VMEM_SHARED((NPAD, D), jnp.float32),
        pltpu.VMEM_SHARED((NPAD, 16), jnp.float32),
        pltpu.SemaphoreType.DMA,
        pltpu.SemaphoreType.DMA,
        pltpu.SemaphoreType.DMA,
        pltpu.SemaphoreType.DMA,
        pltpu.SemaphoreType.DMA,
        pltpu.SemaphoreType.DMA,
        pltpu.SemaphoreType.DMA,
        pltpu.SemaphoreType.DMA,
    ],
    compiler_params=pltpu.CompilerParams(use_tc_tiling_on_sc=False),
)
def _sc_agg(x_hbm, idx_hbm, zrows_hbm, zdeg_hbm, aggf_hbm, dego_hbm,
            idx0, idx1, rows0, rows1, ones, aggsh, degsh,
            semi0, semi1, semg0, semg1, sems0, sems1, semd0, semd1):
    cid = lax.axis_index("c")
    sid = lax.axis_index("s")
    wid = sid * NC + cid
    base = wid * CHUNKS_PER_W

    for j in range(C):
        ones[j, :] = jnp.ones((16,), jnp.float32)

    def start_idx(ci, bank, sem):
        pltpu.async_copy(idx_hbm.at[ci], bank, sem)

    def wait_idx(bank, sem):
        pltpu.make_async_copy(idx_hbm.at[0], bank, sem).wait()

    def start_gather(bank, buf, sem):
        pltpu.async_copy(x_hbm.at[bank.at[0]], buf, sem)

    def wait_gather(buf, sem):
        pltpu.make_async_copy(x_hbm.at[idx0.at[0]], buf, sem).wait()

    def start_scatter(buf, bank, sems, semd):
        pltpu.async_copy(buf, aggsh.at[bank.at[1]], sems, add=True)
        pltpu.async_copy(ones, degsh.at[bank.at[1]], semd, add=True)

    def wait_scatter(buf, sems, semd):
        pltpu.make_async_copy(buf, aggsh.at[idx0.at[1]], sems).wait()
        pltpu.make_async_copy(ones, degsh.at[idx0.at[1]], semd).wait()

    start_idx(base, idx0, semi0)
    start_idx(base + 1, idx1, semi1)
    wait_idx(idx0, semi0)
    start_gather(idx0, rows0, semg0)

    pltpu.sync_copy(zrows_hbm, aggsh.at[pl.ds(sid * ROWS_PER_S, ROWS_PER_S)])
    pltpu.sync_copy(zdeg_hbm, degsh.at[pl.ds(sid * ROWS_PER_S, ROWS_PER_S)])
    plsc.subcore_barrier()

    def _loop_body(k, _):
        c0 = base + 2 * k
        c1 = c0 + 1
        last = k >= CHUNKS_PER_W // 2 - 1
        wait_gather(rows0, semg0)
        start_scatter(rows0, idx0, sems0, semd0)
        wait_idx(idx1, semi1)
        start_gather(idx1, rows1, semg1)
        wait_scatter(rows0, sems0, semd0)

        @pl.when(~last)
        def _():
            start_idx(c0 + 2, idx0, semi0)

        wait_gather(rows1, semg1)
        start_scatter(rows1, idx1, sems1, semd1)

        @pl.when(~last)
        def _():
            wait_idx(idx0, semi0)
            start_gather(idx0, rows0, semg0)

        wait_scatter(rows1, sems1, semd1)

        @pl.when(~last)
        def _():
            start_idx(c1 + 2, idx1, semi1)

        return 0

    lax.fori_loop(0, CHUNKS_PER_W // 2, _loop_body, 0)

    @pl.when(wid < REMAINDER)
    def _():
        ci = NW * CHUNKS_PER_W + wid
        pltpu.sync_copy(idx_hbm.at[ci], idx0)
        pltpu.async_copy(x_hbm.at[idx0.at[0]], rows0, semg0).wait()
        pltpu.sync_copy(rows0, aggsh.at[idx0.at[1]], add=True)
        pltpu.sync_copy(ones, degsh.at[idx0.at[1]], add=True)

    plsc.subcore_barrier()

    pltpu.sync_copy(aggsh.at[pl.ds(sid * ROWS_PER_S, ROWS_PER_S)],
                    aggf_hbm.at[cid, pl.ds(sid * ROWS_PER_S, ROWS_PER_S)])
    pltpu.sync_copy(degsh.at[pl.ds(sid * ROWS_PER_S, ROWS_PER_S)],
                    dego_hbm.at[cid, pl.ds(sid * ROWS_PER_S, ROWS_PER_S)])


_TC_R = 512


def _tc_body(ag_ref, deg_ref, x_ref, w_ref, ws_ref, b_ref, o_ref):
    feat = ag_ref[0] + ag_ref[1]
    deg = deg_ref[0, :, 0:1] + deg_ref[1, :, 0:1]
    m = feat / jnp.maximum(deg, 1.0)
    o_ref[...] = jnp.maximum(
        jnp.dot(m, w_ref[...], preferred_element_type=jnp.float32)
        + jnp.dot(x_ref[...], ws_ref[...], preferred_element_type=jnp.float32)
        + b_ref[...],
        0.0,
    )


def _tc_finish(agg, deg, x, w, ws, b2):
    return pl.pallas_call(
        _tc_body,
        grid=(NPAD // _TC_R,),
        in_specs=[
            pl.BlockSpec((NC, _TC_R, D), lambda i: (0, i, 0)),
            pl.BlockSpec((NC, _TC_R, 16), lambda i: (0, i, 0)),
            pl.BlockSpec((_TC_R, D), lambda i: (i, 0)),
            pl.BlockSpec((D, D), lambda i: (0, 0)),
            pl.BlockSpec((D, D), lambda i: (0, 0)),
            pl.BlockSpec((1, D), lambda i: (0, 0)),
        ],
        out_specs=pl.BlockSpec((_TC_R, D), lambda i: (i, 0)),
        out_shape=jax.ShapeDtypeStruct((N, D), jnp.float32),
    )(agg, deg, x, w, ws, b2)


def kernel(x, edge_index, W, W_self, b):
    idx = edge_index.reshape(2, NCHUNK, C).transpose(1, 0, 2)
    zrows = jnp.zeros((ROWS_PER_S, D), jnp.float32)
    zdeg = jnp.zeros((ROWS_PER_S, 16), jnp.float32)
    aggf, dego = _sc_agg(x, idx, zrows, zdeg)
    return _tc_finish(aggf, dego, x, W, W_self, b.reshape(1, D))


# speedup vs baseline: 1.0402x; 1.0382x over previous
import functools

import jax
import jax.numpy as jnp
from jax import lax
from jax.experimental import pallas as pl
from jax.experimental.pallas import tpu as pltpu
from jax.experimental.pallas import tpu_sc as plsc

N = 10000
E = 320000
D = 128
NPAD = 10240
NC = 2
NS = 16
NW = NC * NS
C = 128
NCHUNK = E // C
CHUNKS_PER_W = NCHUNK // NW
REMAINDER = NCHUNK - CHUNKS_PER_W * NW
ROWS_PER_S = NPAD // NS

_sc_mesh = plsc.VectorSubcoreMesh(core_axis_name="c", subcore_axis_name="s")


@functools.partial(
    pl.kernel,
    out_type=(
        jax.ShapeDtypeStruct((NC, NPAD, D), jnp.float32),
        jax.ShapeDtypeStruct((NC, NPAD, 16), jnp.float32),
    ),
    mesh=_sc_mesh,
    scratch_types=[
        pltpu.VMEM((2, C), jnp.int32),
        pltpu.VMEM((2, C), jnp.int32),
        pltpu.VMEM((C, D), jnp.float32),
        pltpu.VMEM((C, D), jnp.float32),
        pltpu.VMEM((C, 16), jnp.float32),
        pltpu.VMEM_SHARED((NPAD, D), jnp.float32),
        pltpu.VMEM_SHARED((NPAD, 16), jnp.float32),
        pltpu.SemaphoreType.DMA,
        pltpu.SemaphoreType.DMA,
        pltpu.SemaphoreType.DMA,
        pltpu.SemaphoreType.DMA,
        pltpu.SemaphoreType.DMA,
        pltpu.SemaphoreType.DMA,
        pltpu.SemaphoreType.DMA,
        pltpu.SemaphoreType.DMA,
    ],
    compiler_params=pltpu.CompilerParams(use_tc_tiling_on_sc=False),
)
def _sc_agg(x_hbm, idx_hbm, zrows_hbm, zdeg_hbm, aggf_hbm, dego_hbm,
            idx0, idx1, rows0, rows1, ones, aggsh, degsh,
            semi0, semi1, semg0, semg1, sems0, sems1, semd0, semd1):
    cid = lax.axis_index("c")
    sid = lax.axis_index("s")
    wid = sid * NC + cid
    base = wid * CHUNKS_PER_W

    for j in range(C):
        ones[j, :] = jnp.ones((16,), jnp.float32)

    def start_idx(ci, bank, sem):
        pltpu.async_copy(idx_hbm.at[ci], bank, sem)

    def wait_idx(bank, sem):
        pltpu.make_async_copy(idx_hbm.at[0], bank, sem).wait()

    def start_gather(bank, buf, sem):
        pltpu.async_copy(x_hbm.at[bank.at[0]], buf, sem)

    def wait_gather(buf, sem):
        pltpu.make_async_copy(x_hbm.at[idx0.at[0]], buf, sem).wait()

    def start_scatter(buf, bank, sems, semd):
        pltpu.async_copy(buf, aggsh.at[bank.at[1]], sems, add=True)
        pltpu.async_copy(ones, degsh.at[bank.at[1]], semd, add=True)

    def wait_scatter(buf, sems, semd):
        pltpu.make_async_copy(buf, aggsh.at[idx0.at[1]], sems).wait()
        pltpu.make_async_copy(ones, degsh.at[idx0.at[1]], semd).wait()

    start_idx(base, idx0, semi0)
    start_idx(base + 1, idx1, semi1)
    wait_idx(idx0, semi0)
    start_gather(idx0, rows0, semg0)

    pltpu.sync_copy(zrows_hbm, aggsh.at[pl.ds(sid * ROWS_PER_S, ROWS_PER_S)])
    pltpu.sync_copy(zdeg_hbm, degsh.at[pl.ds(sid * ROWS_PER_S, ROWS_PER_S)])
    plsc.subcore_barrier()

    def _loop_body(k, _):
        c0 = base + 2 * k
        c1 = c0 + 1
        last = k >= CHUNKS_PER_W // 2 - 1
        wait_gather(rows0, semg0)
        start_scatter(rows0, idx0, sems0, semd0)
        wait_idx(idx1, semi1)
        start_gather(idx1, rows1, semg1)
        wait_scatter(rows0, sems0, semd0)

        @pl.when(~last)
        def _():
            start_idx(c0 + 2, idx0, semi0)

        wait_gather(rows1, semg1)
        start_scatter(rows1, idx1, sems1, semd1)

        @pl.when(~last)
        def _():
            wait_idx(idx0, semi0)
            start_gather(idx0, rows0, semg0)

        wait_scatter(rows1, sems1, semd1)

        @pl.when(~last)
        def _():
            start_idx(c1 + 2, idx1, semi1)

        return 0

    lax.fori_loop(0, CHUNKS_PER_W // 2, _loop_body, 0)

    @pl.when(wid < REMAINDER)
    def _():
        ci = NW * CHUNKS_PER_W + wid
        pltpu.sync_copy(idx_hbm.at[ci], idx0)
        pltpu.async_copy(x_hbm.at[idx0.at[0]], rows0, semg0).wait()
        pltpu.sync_copy(rows0, aggsh.at[idx0.at[1]], add=True)
        pltpu.sync_copy(ones, degsh.at[idx0.at[1]], add=True)

    plsc.subcore_barrier()

    pltpu.sync_copy(aggsh.at[pl.ds(sid * ROWS_PER_S, ROWS_PER_S)],
                    aggf_hbm.at[cid, pl.ds(sid * ROWS_PER_S, ROWS_PER_S)])
    pltpu.sync_copy(degsh.at[pl.ds(sid * ROWS_PER_S, ROWS_PER_S)],
                    dego_hbm.at[cid, pl.ds(sid * ROWS_PER_S, ROWS_PER_S)])


_TC_R = 1024


def _tc_body(ag_ref, deg_ref, x_ref, w_ref, ws_ref, b_ref, o_ref):
    feat = ag_ref[0] + ag_ref[1]
    deg = deg_ref[0, :, 0:1] + deg_ref[1, :, 0:1]
    m = feat / jnp.maximum(deg, 1.0)
    o_ref[...] = jnp.maximum(
        jnp.dot(m, w_ref[...], preferred_element_type=jnp.float32)
        + jnp.dot(x_ref[...], ws_ref[...], preferred_element_type=jnp.float32)
        + b_ref[...],
        0.0,
    )


def _tc_finish(agg, deg, x, w, ws, b2):
    return pl.pallas_call(
        _tc_body,
        grid=(NPAD // _TC_R,),
        in_specs=[
            pl.BlockSpec((NC, _TC_R, D), lambda i: (0, i, 0)),
            pl.BlockSpec((NC, _TC_R, 16), lambda i: (0, i, 0)),
            pl.BlockSpec((_TC_R, D), lambda i: (i, 0)),
            pl.BlockSpec((D, D), lambda i: (0, 0)),
            pl.BlockSpec((D, D), lambda i: (0, 0)),
            pl.BlockSpec((1, D), lambda i: (0, 0)),
        ],
        out_specs=pl.BlockSpec((_TC_R, D), lambda i: (i, 0)),
        out_shape=jax.ShapeDtypeStruct((N, D), jnp.float32),
    )(agg, deg, x, w, ws, b2)


def kernel(x, edge_index, W, W_self, b):
    idx = edge_index.reshape(2, NCHUNK, C).transpose(1, 0, 2)
    zrows = jnp.zeros((ROWS_PER_S, D), jnp.float32)
    zdeg = jnp.zeros((ROWS_PER_S, 16), jnp.float32)
    aggf, dego = _sc_agg(x, idx, zrows, zdeg)
    return _tc_finish(aggf, dego, x, W, W_self, b.reshape(1, D))
